# baseline (speedup 1.0000x reference)
import jax
import jax.numpy as jnp
from jax import lax
from jax.experimental import pallas as pl
from jax.experimental.pallas import tpu as pltpu
from jax.experimental.pallas import tpu_sc as plsc

B, L = 4096, 200
EMB_DIM = 64
N_DISCRETE = 10
OUT_DIM = 5 * N_DISCRETE

NC, NS = 2, 16
NW = NC * NS
BPW = B // NW
NCH = EMB_DIM // 16
NGC = BPW // 16
NBUF = 8


def _pool_body(tokens_hbm, emb_hbm, out_hbm, tok_v, idx_v, rows_v, out_v,
               emb0_v, idx0_v, *sems):
    wid = lax.axis_index("s") * NC + lax.axis_index("c")
    base = wid * BPW

    zeros16i = jnp.zeros((16,), jnp.int32)
    zeros16f = jnp.zeros((16,), jnp.float32)

    idx0_v[pl.ds(0, 16)] = zeros16i
    pltpu.async_copy(emb_hbm.at[idx0_v], emb0_v, sems[0]).wait()

    pltpu.sync_copy(tokens_hbm.at[:, wid], tok_v)

    def zero_body(g, _):
        for c in range(NCH):
            out_v[g, pl.ds(c * 16, 16)] = zeros16f
        return 0

    lax.fori_loop(0, BPW, zero_body, 0)

    def fire(l, b):
        for gc in range(NGC):
            t = tok_v[l // 8, l % 8, pl.ds(gc * 16, 16)]
            sig = ((t & jnp.int32(-4096))
                   | ((t & jnp.int32(2047)) << 1)
                   | ((t & jnp.int32(4095)) >> 11))
            idx_v[b, pl.ds(gc * 16, 16)] = sig
        pltpu.async_copy(emb_hbm.at[idx_v.at[b]], rows_v.at[b], sems[b])

    def wait(b):
        pltpu.make_async_copy(emb_hbm.at[idx_v.at[b]], rows_v.at[b],
                              sems[b]).wait()

    def accumulate(l, b, cnts):
        def acc_body(j, _):
            for u in range(4):
                i = j * 4 + u
                for c in range(NCH):
                    plsc.addupdate(out_v.at[i, pl.ds(c * 16, 16)],
                                   rows_v[b, i, pl.ds(c * 16, 16)])
            return 0

        lax.fori_loop(0, BPW // 4, acc_body, 0)

        new = []
        for gc in range(NGC):
            tok = tok_v[l // 8, l % 8, pl.ds(gc * 16, 16)]
            new.append(cnts[gc]
                       + jnp.where(tok == 0, 1, 0).astype(jnp.int32))
        return tuple(new)

    for l in range(NBUF - 1):
        fire(l, l)

    cnts0 = tuple(zeros16i for _ in range(NGC))

    def outer(i, cnts):
        for b in range(NBUF):
            l = NBUF * i + b

            @pl.when(l + NBUF - 1 < L)
            def _():
                fire(l + NBUF - 1, (b + NBUF - 1) % NBUF)

            wait(b)
            cnts = accumulate(l, b, cnts)
        return cnts

    cnts = lax.fori_loop(0, L // NBUF, outer, cnts0)

    for gc in range(NGC):
        cnt_vec = cnts[gc]
        for u in range(16):
            g = gc * 16 + u
            n_pad = cnt_vec[u]
            npad_v = jnp.full((16,), n_pad, jnp.int32).astype(jnp.float32)
            inv_v = jnp.float32(1.0) / (jnp.float32(L) - npad_v)
            for c in range(NCH):
                emb0_c = emb0_v[0, pl.ds(c * 16, 16)]
                acc = out_v[g, pl.ds(c * 16, 16)]
                out_v[g, pl.ds(c * 16, 16)] = (acc - npad_v * emb0_c) * inv_v

    pltpu.sync_copy(out_v, out_hbm.at[pl.ds(base, BPW)])


def _masked_mean_pool(tokens_t, emb):
    mesh = plsc.VectorSubcoreMesh(core_axis_name="c", subcore_axis_name="s")
    kern = pl.kernel(
        _pool_body,
        out_type=jax.ShapeDtypeStruct((B, EMB_DIM), jnp.float32),
        mesh=mesh,
        scratch_types=[
            pltpu.VMEM((L // 8, 8, BPW), jnp.int32),
            pltpu.VMEM((NBUF, BPW), jnp.int32),
            pltpu.VMEM((NBUF, BPW, EMB_DIM), jnp.float32),
            pltpu.VMEM((BPW, EMB_DIM), jnp.float32),
            pltpu.VMEM((16, EMB_DIM), jnp.float32),
            pltpu.VMEM((16,), jnp.int32),
        ] + [pltpu.SemaphoreType.DMA] * NBUF,
        compiler_params=pltpu.CompilerParams(use_tc_tiling_on_sc=False),
    )
    return kern(tokens_t, emb)


VOCAB = 1000000
DT_W = 4096
DT_H = DT_W // 2
DT_STEPS = -(-VOCAB // DT_W)
VOCAB_PAD = DT_STEPS * DT_W


def _detile_body(embt_ref, out_ref):
    x = embt_ref[...]
    eye = (lax.broadcasted_iota(jnp.int32, (EMB_DIM, EMB_DIM), 0)
           == lax.broadcasted_iota(jnp.int32, (EMB_DIM, EMB_DIM), 1)
           ).astype(jnp.float32)
    dn = (((0,), (0,)), ((), ()))
    ta = lax.dot_general(x[:, :DT_H], eye, dn,
                         precision=lax.Precision.HIGHEST,
                         preferred_element_type=jnp.float32)
    tb = lax.dot_general(x[:, DT_H:], eye, dn,
                         precision=lax.Precision.HIGHEST,
                         preferred_element_type=jnp.float32)
    out_ref[...] = jnp.concatenate([ta, tb], axis=1)


def _detile(emb_t):
    return pl.pallas_call(
        _detile_body,
        grid=(DT_STEPS,),
        in_specs=[pl.BlockSpec((EMB_DIM, DT_W), lambda k: (0, k))],
        out_specs=pl.BlockSpec((DT_H, 2 * EMB_DIM), lambda k: (k, 0)),
        out_shape=jax.ShapeDtypeStruct((VOCAB_PAD // 2, 2 * EMB_DIM),
                                       jnp.float32),
    )(emb_t)


def _mlp_body(avg_ref, w1_ref, b1_ref, w2_ref, b2_ref, out_ref):
    h = jnp.dot(avg_ref[...], w1_ref[...], preferred_element_type=jnp.float32)
    h = jnp.maximum(h + b1_ref[...], 0.0)
    out_ref[...] = (
        jnp.dot(h, w2_ref[...], preferred_element_type=jnp.float32)
        + b2_ref[...]
    )


def _mlp(avg, W1, b1, W2, b2):
    return pl.pallas_call(
        _mlp_body,
        out_shape=jax.ShapeDtypeStruct((B, OUT_DIM), jnp.float32),
    )(avg, W1, b1.reshape(1, -1), W2, b2.reshape(1, -1))


def kernel(tokens, emb, W1, b1, W2, b2):
    tokens_tiled = tokens.reshape(B // 128, 128, L // 8, 8).transpose(2, 0, 3, 1)
    emb_lin = _detile(emb.T).reshape(VOCAB_PAD, EMB_DIM)
    avg = _masked_mean_pool(tokens_tiled, emb_lin)
    logits = _mlp(avg, W1, b1, W2, b2)
    return logits.reshape(B, OUT_DIM // N_DISCRETE, N_DISCRETE)


# speedup vs baseline: 1.4067x; 1.4067x over previous
import jax
import jax.numpy as jnp
from jax import lax
from jax.experimental import pallas as pl
from jax.experimental.pallas import tpu as pltpu
from jax.experimental.pallas import tpu_sc as plsc

B, L = 4096, 200
EMB_DIM = 64
N_DISCRETE = 10
OUT_DIM = 5 * N_DISCRETE

NC, NS = 2, 16
NW = NC * NS
BPW = B // NW
NCH = EMB_DIM // 16
NGC = BPW // 16
NBUF = 8


def _pool_body(tokens_hbm, emb_hbm, out_hbm, tok_v, idx_v, rows_v, out_v,
               emb0_v, idx0_v, *sems):
    wid = lax.axis_index("s") * NC + lax.axis_index("c")
    base = wid * BPW

    zeros16i = jnp.zeros((16,), jnp.int32)
    zeros16f = jnp.zeros((16,), jnp.float32)

    idx0_v[pl.ds(0, 16)] = zeros16i
    pltpu.async_copy(emb_hbm.at[idx0_v], emb0_v, sems[0]).wait()

    pltpu.sync_copy(tokens_hbm.at[:, wid], tok_v)

    def zero_body(g, _):
        for c in range(NCH):
            out_v[g, pl.ds(c * 16, 16)] = zeros16f
        return 0

    lax.fori_loop(0, BPW, zero_body, 0)

    def fire(l, b):
        for gc in range(NGC):
            t = tok_v[l // 8, l % 8, pl.ds(gc * 16, 16)]
            sig = ((t & jnp.int32(-4096))
                   | ((t & jnp.int32(2047)) << 1)
                   | ((t & jnp.int32(4095)) >> 11))
            idx_v[b, pl.ds(gc * 16, 16)] = sig
        pltpu.async_copy(emb_hbm.at[idx_v.at[b]], rows_v.at[b], sems[b])

    def wait(b):
        pltpu.make_async_copy(emb_hbm.at[idx_v.at[b]], rows_v.at[b],
                              sems[b]).wait()

    def accumulate(l, b, cnts):
        def acc_body(j, _):
            for u in range(4):
                i = j * 4 + u
                for c in range(NCH):
                    plsc.addupdate(out_v.at[i, pl.ds(c * 16, 16)],
                                   rows_v[b, i, pl.ds(c * 16, 16)])
            return 0

        lax.fori_loop(0, BPW // 4, acc_body, 0)

        new = []
        for gc in range(NGC):
            tok = tok_v[l // 8, l % 8, pl.ds(gc * 16, 16)]
            new.append(cnts[gc]
                       + jnp.where(tok == 0, 1, 0).astype(jnp.int32))
        return tuple(new)

    for l in range(NBUF - 1):
        fire(l, l)

    cnts0 = tuple(zeros16i for _ in range(NGC))

    def outer(i, cnts):
        for b in range(NBUF):
            l = NBUF * i + b

            @pl.when(l + NBUF - 1 < L)
            def _():
                fire(l + NBUF - 1, (b + NBUF - 1) % NBUF)

            wait(b)
            cnts = accumulate(l, b, cnts)
        return cnts

    cnts = lax.fori_loop(0, L // NBUF, outer, cnts0)

    for gc in range(NGC):
        cnt_vec = cnts[gc]
        for u in range(16):
            g = gc * 16 + u
            n_pad = cnt_vec[u]
            npad_v = jnp.full((16,), n_pad, jnp.int32).astype(jnp.float32)
            inv_v = jnp.float32(1.0) / (jnp.float32(L) - npad_v)
            for c in range(NCH):
                emb0_c = emb0_v[0, pl.ds(c * 16, 16)]
                acc = out_v[g, pl.ds(c * 16, 16)]
                out_v[g, pl.ds(c * 16, 16)] = (acc - npad_v * emb0_c) * inv_v

    pltpu.sync_copy(out_v, out_hbm.at[pl.ds(base, BPW)])


def _masked_mean_pool(tokens_t, emb):
    mesh = plsc.VectorSubcoreMesh(core_axis_name="c", subcore_axis_name="s")
    kern = pl.kernel(
        _pool_body,
        out_type=jax.ShapeDtypeStruct((B, EMB_DIM), jnp.float32),
        mesh=mesh,
        scratch_types=[
            pltpu.VMEM((L // 8, 8, BPW), jnp.int32),
            pltpu.VMEM((NBUF, BPW), jnp.int32),
            pltpu.VMEM((NBUF, BPW, EMB_DIM), jnp.float32),
            pltpu.VMEM((BPW, EMB_DIM), jnp.float32),
            pltpu.VMEM((16, EMB_DIM), jnp.float32),
            pltpu.VMEM((16,), jnp.int32),
        ] + [pltpu.SemaphoreType.DMA] * NBUF,
        compiler_params=pltpu.CompilerParams(use_tc_tiling_on_sc=False),
    )
    return kern(tokens_t, emb)


VOCAB = 1000000
DT_W = 4096
DT_H = DT_W // 2
DT_STEPS = -(-VOCAB // DT_W)
VOCAB_PAD = DT_STEPS * DT_W


def _detile_body(embt_ref, out_ref):
    x = embt_ref[...]
    ta = jnp.swapaxes(x[:, :DT_H], 0, 1)
    tb = jnp.swapaxes(x[:, DT_H:], 0, 1)
    out_ref[...] = jnp.concatenate([ta, tb], axis=1)


def _detile(emb_t):
    return pl.pallas_call(
        _detile_body,
        grid=(DT_STEPS,),
        in_specs=[pl.BlockSpec((EMB_DIM, DT_W), lambda k: (0, k))],
        out_specs=pl.BlockSpec((DT_H, 2 * EMB_DIM), lambda k: (k, 0)),
        out_shape=jax.ShapeDtypeStruct((VOCAB_PAD // 2, 2 * EMB_DIM),
                                       jnp.float32),
    )(emb_t)


def _mlp_body(avg_ref, w1_ref, b1_ref, w2_ref, b2_ref, out_ref):
    h = jnp.dot(avg_ref[...], w1_ref[...], preferred_element_type=jnp.float32)
    h = jnp.maximum(h + b1_ref[...], 0.0)
    out_ref[...] = (
        jnp.dot(h, w2_ref[...], preferred_element_type=jnp.float32)
        + b2_ref[...]
    )


def _mlp(avg, W1, b1, W2, b2):
    return pl.pallas_call(
        _mlp_body,
        out_shape=jax.ShapeDtypeStruct((B, OUT_DIM), jnp.float32),
    )(avg, W1, b1.reshape(1, -1), W2, b2.reshape(1, -1))


def kernel(tokens, emb, W1, b1, W2, b2):
    tokens_tiled = tokens.reshape(B // 128, 128, L // 8, 8).transpose(2, 0, 3, 1)
    emb_lin = _detile(emb.T).reshape(VOCAB_PAD, EMB_DIM)
    avg = _masked_mean_pool(tokens_tiled, emb_lin)
    logits = _mlp(avg, W1, b1, W2, b2)
    return logits.reshape(B, OUT_DIM // N_DISCRETE, N_DISCRETE)


# speedup vs baseline: 1.6265x; 1.1562x over previous
import jax
import jax.numpy as jnp
from jax import lax
from jax.experimental import pallas as pl
from jax.experimental.pallas import tpu as pltpu
from jax.experimental.pallas import tpu_sc as plsc

B, L = 4096, 200
EMB_DIM = 64
N_DISCRETE = 10
OUT_DIM = 5 * N_DISCRETE

NC, NS = 2, 16
NW = NC * NS
BPW = B // NW
NCH = EMB_DIM // 16
NGC = BPW // 16
NBUF = 8


def _pool_body(tokens_hbm, emb_hbm, out_hbm, tok_v, idx_v, rows_v, out_v,
               emb0_v, idx0_v, *sems):
    wid = lax.axis_index("s") * NC + lax.axis_index("c")
    base = wid * BPW

    zeros16i = jnp.zeros((16,), jnp.int32)
    zeros16f = jnp.zeros((16,), jnp.float32)

    idx0_v[pl.ds(0, 16)] = zeros16i
    pltpu.async_copy(emb_hbm.at[idx0_v], emb0_v, sems[0]).wait()

    pltpu.sync_copy(tokens_hbm.at[:, wid], tok_v)

    def zero_body(g, _):
        for c in range(NCH):
            out_v[g, pl.ds(c * 16, 16)] = zeros16f
        return 0

    lax.fori_loop(0, BPW, zero_body, 0)

    def fire(l, b):
        for gc in range(NGC):
            t = tok_v[l // 8, l % 8, pl.ds(gc * 16, 16)]
            sig = ((t & jnp.int32(-DT_W))
                   | ((t & jnp.int32(DT_H - 1)) << 1)
                   | ((t & jnp.int32(DT_W - 1)) >> 12))
            idx_v[b, pl.ds(gc * 16, 16)] = sig
        pltpu.async_copy(emb_hbm.at[idx_v.at[b]], rows_v.at[b], sems[b])

    def wait(b):
        pltpu.make_async_copy(emb_hbm.at[idx_v.at[b]], rows_v.at[b],
                              sems[b]).wait()

    def accumulate(l, b, cnts):
        def acc_body(j, _):
            for u in range(4):
                i = j * 4 + u
                for c in range(NCH):
                    plsc.addupdate(out_v.at[i, pl.ds(c * 16, 16)],
                                   rows_v[b, i, pl.ds(c * 16, 16)])
            return 0

        lax.fori_loop(0, BPW // 4, acc_body, 0)

        new = []
        for gc in range(NGC):
            tok = tok_v[l // 8, l % 8, pl.ds(gc * 16, 16)]
            new.append(cnts[gc]
                       + jnp.where(tok == 0, 1, 0).astype(jnp.int32))
        return tuple(new)

    for l in range(NBUF - 1):
        fire(l, l)

    cnts0 = tuple(zeros16i for _ in range(NGC))

    def outer(i, cnts):
        for b in range(NBUF):
            l = NBUF * i + b

            @pl.when(l + NBUF - 1 < L)
            def _():
                fire(l + NBUF - 1, (b + NBUF - 1) % NBUF)

            wait(b)
            cnts = accumulate(l, b, cnts)
        return cnts

    cnts = lax.fori_loop(0, L // NBUF, outer, cnts0)

    for gc in range(NGC):
        cnt_vec = cnts[gc]
        for u in range(16):
            g = gc * 16 + u
            n_pad = cnt_vec[u]
            npad_v = jnp.full((16,), n_pad, jnp.int32).astype(jnp.float32)
            inv_v = jnp.float32(1.0) / (jnp.float32(L) - npad_v)
            for c in range(NCH):
                emb0_c = emb0_v[0, pl.ds(c * 16, 16)]
                acc = out_v[g, pl.ds(c * 16, 16)]
                out_v[g, pl.ds(c * 16, 16)] = (acc - npad_v * emb0_c) * inv_v

    pltpu.sync_copy(out_v, out_hbm.at[pl.ds(base, BPW)])


def _masked_mean_pool(tokens_t, emb):
    mesh = plsc.VectorSubcoreMesh(core_axis_name="c", subcore_axis_name="s")
    kern = pl.kernel(
        _pool_body,
        out_type=jax.ShapeDtypeStruct((B, EMB_DIM), jnp.float32),
        mesh=mesh,
        scratch_types=[
            pltpu.VMEM((L // 8, 8, BPW), jnp.int32),
            pltpu.VMEM((NBUF, BPW), jnp.int32),
            pltpu.VMEM((NBUF, BPW, EMB_DIM), jnp.float32),
            pltpu.VMEM((BPW, EMB_DIM), jnp.float32),
            pltpu.VMEM((16, EMB_DIM), jnp.float32),
            pltpu.VMEM((16,), jnp.int32),
        ] + [pltpu.SemaphoreType.DMA] * NBUF,
        compiler_params=pltpu.CompilerParams(use_tc_tiling_on_sc=False),
    )
    return kern(tokens_t, emb)


VOCAB = 1000000
DT_W = 8192
DT_H = DT_W // 2
DT_STEPS = -(-VOCAB // DT_W)
VOCAB_PAD = DT_STEPS * DT_W


def _detile_body(embt_ref, out_ref):
    x = embt_ref[...]
    ta = jnp.swapaxes(x[:, :DT_H], 0, 1)
    tb = jnp.swapaxes(x[:, DT_H:], 0, 1)
    out_ref[...] = jnp.concatenate([ta, tb], axis=1)


def _detile(emb_t):
    return pl.pallas_call(
        _detile_body,
        grid=(DT_STEPS,),
        in_specs=[pl.BlockSpec((EMB_DIM, DT_W), lambda k: (0, k))],
        out_specs=pl.BlockSpec((DT_H, 2 * EMB_DIM), lambda k: (k, 0)),
        out_shape=jax.ShapeDtypeStruct((VOCAB_PAD // 2, 2 * EMB_DIM),
                                       jnp.float32),
    )(emb_t)


def _mlp_body(avg_ref, w1_ref, b1_ref, w2_ref, b2_ref, out_ref):
    h = jnp.dot(avg_ref[...], w1_ref[...], preferred_element_type=jnp.float32)
    h = jnp.maximum(h + b1_ref[...], 0.0)
    out_ref[...] = (
        jnp.dot(h, w2_ref[...], preferred_element_type=jnp.float32)
        + b2_ref[...]
    )


def _mlp(avg, W1, b1, W2, b2):
    return pl.pallas_call(
        _mlp_body,
        out_shape=jax.ShapeDtypeStruct((B, OUT_DIM), jnp.float32),
    )(avg, W1, b1.reshape(1, -1), W2, b2.reshape(1, -1))


def kernel(tokens, emb, W1, b1, W2, b2):
    tokens_tiled = tokens.reshape(B // 128, 128, L // 8, 8).transpose(2, 0, 3, 1)
    emb_lin = _detile(emb.T).reshape(VOCAB_PAD, EMB_DIM)
    avg = _masked_mean_pool(tokens_tiled, emb_lin)
    logits = _mlp(avg, W1, b1, W2, b2)
    return logits.reshape(B, OUT_DIM // N_DISCRETE, N_DISCRETE)


# speedup vs baseline: 1.7612x; 1.0828x over previous
import jax
import jax.numpy as jnp
from jax import lax
from jax.experimental import pallas as pl
from jax.experimental.pallas import tpu as pltpu
from jax.experimental.pallas import tpu_sc as plsc

B, L = 4096, 200
EMB_DIM = 64
N_DISCRETE = 10
OUT_DIM = 5 * N_DISCRETE

NC, NS = 2, 16
NW = NC * NS
BPW = B // NW
NCH = EMB_DIM // 16
NGC = BPW // 16
NBUF = 8


def _pool_body(tokens_hbm, emb_hbm, out_hbm, tok_v, idx_v, rows_v, out_v,
               emb0_v, idx0_v, *sems):
    wid = lax.axis_index("s") * NC + lax.axis_index("c")
    base = wid * BPW

    zeros16i = jnp.zeros((16,), jnp.int32)
    zeros16f = jnp.zeros((16,), jnp.float32)

    idx0_v[pl.ds(0, 16)] = zeros16i
    pltpu.async_copy(emb_hbm.at[idx0_v], emb0_v, sems[0]).wait()

    pltpu.sync_copy(tokens_hbm.at[:, wid], tok_v)

    def zero_body(g, _):
        for c in range(NCH):
            out_v[g, pl.ds(c * 16, 16)] = zeros16f
        return 0

    lax.fori_loop(0, BPW, zero_body, 0)

    def fire(l, b):
        for gc in range(NGC):
            t = tok_v[l // 8, l % 8, pl.ds(gc * 16, 16)]
            sig = ((t & jnp.int32(-DT_W))
                   | ((t & jnp.int32(DT_H - 1)) << 1)
                   | ((t & jnp.int32(DT_W - 1)) >> 13))
            idx_v[b, pl.ds(gc * 16, 16)] = sig
        pltpu.async_copy(emb_hbm.at[idx_v.at[b]], rows_v.at[b], sems[b])

    def wait(b):
        pltpu.make_async_copy(emb_hbm.at[idx_v.at[b]], rows_v.at[b],
                              sems[b]).wait()

    def accumulate(l, b, cnts):
        def acc_body(j, _):
            for u in range(4):
                i = j * 4 + u
                for c in range(NCH):
                    plsc.addupdate(out_v.at[i, pl.ds(c * 16, 16)],
                                   rows_v[b, i, pl.ds(c * 16, 16)])
            return 0

        lax.fori_loop(0, BPW // 4, acc_body, 0)

        new = []
        for gc in range(NGC):
            tok = tok_v[l // 8, l % 8, pl.ds(gc * 16, 16)]
            new.append(cnts[gc]
                       + jnp.where(tok == 0, 1, 0).astype(jnp.int32))
        return tuple(new)

    for l in range(NBUF - 1):
        fire(l, l)

    cnts0 = tuple(zeros16i for _ in range(NGC))

    def outer(i, cnts):
        for b in range(NBUF):
            l = NBUF * i + b

            @pl.when(l + NBUF - 1 < L)
            def _():
                fire(l + NBUF - 1, (b + NBUF - 1) % NBUF)

            wait(b)
            cnts = accumulate(l, b, cnts)
        return cnts

    cnts = lax.fori_loop(0, L // NBUF, outer, cnts0)

    for gc in range(NGC):
        cnt_vec = cnts[gc]
        for u in range(16):
            g = gc * 16 + u
            n_pad = cnt_vec[u]
            npad_v = jnp.full((16,), n_pad, jnp.int32).astype(jnp.float32)
            inv_v = jnp.float32(1.0) / (jnp.float32(L) - npad_v)
            for c in range(NCH):
                emb0_c = emb0_v[0, pl.ds(c * 16, 16)]
                acc = out_v[g, pl.ds(c * 16, 16)]
                out_v[g, pl.ds(c * 16, 16)] = (acc - npad_v * emb0_c) * inv_v

    pltpu.sync_copy(out_v, out_hbm.at[pl.ds(base, BPW)])


def _masked_mean_pool(tokens_t, emb):
    mesh = plsc.VectorSubcoreMesh(core_axis_name="c", subcore_axis_name="s")
    kern = pl.kernel(
        _pool_body,
        out_type=jax.ShapeDtypeStruct((B, EMB_DIM), jnp.float32),
        mesh=mesh,
        scratch_types=[
            pltpu.VMEM((L // 8, 8, BPW), jnp.int32),
            pltpu.VMEM((NBUF, BPW), jnp.int32),
            pltpu.VMEM((NBUF, BPW, EMB_DIM), jnp.float32),
            pltpu.VMEM((BPW, EMB_DIM), jnp.float32),
            pltpu.VMEM((16, EMB_DIM), jnp.float32),
            pltpu.VMEM((16,), jnp.int32),
        ] + [pltpu.SemaphoreType.DMA] * NBUF,
        compiler_params=pltpu.CompilerParams(use_tc_tiling_on_sc=False),
    )
    return kern(tokens_t, emb)


VOCAB = 1000000
DT_W = 16384
DT_H = DT_W // 2
DT_STEPS = -(-VOCAB // DT_W)
VOCAB_PAD = DT_STEPS * DT_W


def _detile_body(embt_ref, out_ref):
    x = embt_ref[...]
    ta = jnp.swapaxes(x[:, :DT_H], 0, 1)
    tb = jnp.swapaxes(x[:, DT_H:], 0, 1)
    out_ref[...] = jnp.concatenate([ta, tb], axis=1)


def _detile(emb_t):
    return pl.pallas_call(
        _detile_body,
        grid=(DT_STEPS,),
        in_specs=[pl.BlockSpec((EMB_DIM, DT_W), lambda k: (0, k))],
        out_specs=pl.BlockSpec((DT_H, 2 * EMB_DIM), lambda k: (k, 0)),
        out_shape=jax.ShapeDtypeStruct((VOCAB_PAD // 2, 2 * EMB_DIM),
                                       jnp.float32),
    )(emb_t)


def _mlp_body(avg_ref, w1_ref, b1_ref, w2_ref, b2_ref, out_ref):
    h = jnp.dot(avg_ref[...], w1_ref[...], preferred_element_type=jnp.float32)
    h = jnp.maximum(h + b1_ref[...], 0.0)
    out_ref[...] = (
        jnp.dot(h, w2_ref[...], preferred_element_type=jnp.float32)
        + b2_ref[...]
    )


def _mlp(avg, W1, b1, W2, b2):
    return pl.pallas_call(
        _mlp_body,
        out_shape=jax.ShapeDtypeStruct((B, OUT_DIM), jnp.float32),
    )(avg, W1, b1.reshape(1, -1), W2, b2.reshape(1, -1))


def kernel(tokens, emb, W1, b1, W2, b2):
    tokens_tiled = tokens.reshape(B // 128, 128, L // 8, 8).transpose(2, 0, 3, 1)
    emb_lin = _detile(emb.T).reshape(VOCAB_PAD, EMB_DIM)
    avg = _masked_mean_pool(tokens_tiled, emb_lin)
    logits = _mlp(avg, W1, b1, W2, b2)
    return logits.reshape(B, OUT_DIM // N_DISCRETE, N_DISCRETE)


# speedup vs baseline: 1.8191x; 1.0329x over previous
import jax
import jax.numpy as jnp
from jax import lax
from jax.experimental import pallas as pl
from jax.experimental.pallas import tpu as pltpu
from jax.experimental.pallas import tpu_sc as plsc

B, L = 4096, 200
EMB_DIM = 64
N_DISCRETE = 10
OUT_DIM = 5 * N_DISCRETE

NC, NS = 2, 16
NW = NC * NS
BPW = B // NW
NCH = EMB_DIM // 16
NGC = BPW // 16
NBUF = 8


def _pool_body(tokens_hbm, emb_hbm, out_hbm, tok_v, idx_v, rows_v, out_v,
               emb0_v, idx0_v, *sems):
    wid = lax.axis_index("s") * NC + lax.axis_index("c")
    base = wid * BPW

    zeros16i = jnp.zeros((16,), jnp.int32)
    zeros16f = jnp.zeros((16,), jnp.float32)

    idx0_v[pl.ds(0, 16)] = zeros16i
    pltpu.async_copy(emb_hbm.at[idx0_v], emb0_v, sems[0]).wait()

    pltpu.sync_copy(tokens_hbm.at[:, wid], tok_v)

    def zero_body(g, _):
        for c in range(NCH):
            out_v[g, pl.ds(c * 16, 16)] = zeros16f
        return 0

    lax.fori_loop(0, BPW, zero_body, 0)

    def fire(l, b):
        for gc in range(NGC):
            t = tok_v[l // 8, l % 8, pl.ds(gc * 16, 16)]
            sig = ((t & jnp.int32(-DT_W))
                   | ((t & jnp.int32(DT_H - 1)) << 1)
                   | ((t & jnp.int32(DT_W - 1)) >> 14))
            idx_v[b, pl.ds(gc * 16, 16)] = sig
        pltpu.async_copy(emb_hbm.at[idx_v.at[b]], rows_v.at[b], sems[b])

    def wait(b):
        pltpu.make_async_copy(emb_hbm.at[idx_v.at[b]], rows_v.at[b],
                              sems[b]).wait()

    def accumulate(l, b, cnts):
        def acc_body(j, _):
            for u in range(4):
                i = j * 4 + u
                for c in range(NCH):
                    plsc.addupdate(out_v.at[i, pl.ds(c * 16, 16)],
                                   rows_v[b, i, pl.ds(c * 16, 16)])
            return 0

        lax.fori_loop(0, BPW // 4, acc_body, 0)

        new = []
        for gc in range(NGC):
            tok = tok_v[l // 8, l % 8, pl.ds(gc * 16, 16)]
            new.append(cnts[gc]
                       + jnp.where(tok == 0, 1, 0).astype(jnp.int32))
        return tuple(new)

    for l in range(NBUF - 1):
        fire(l, l)

    cnts0 = tuple(zeros16i for _ in range(NGC))

    def outer(i, cnts):
        for b in range(NBUF):
            l = NBUF * i + b

            @pl.when(l + NBUF - 1 < L)
            def _():
                fire(l + NBUF - 1, (b + NBUF - 1) % NBUF)

            wait(b)
            cnts = accumulate(l, b, cnts)
        return cnts

    cnts = lax.fori_loop(0, L // NBUF, outer, cnts0)

    for gc in range(NGC):
        cnt_vec = cnts[gc]
        for u in range(16):
            g = gc * 16 + u
            n_pad = cnt_vec[u]
            npad_v = jnp.full((16,), n_pad, jnp.int32).astype(jnp.float32)
            inv_v = jnp.float32(1.0) / (jnp.float32(L) - npad_v)
            for c in range(NCH):
                emb0_c = emb0_v[0, pl.ds(c * 16, 16)]
                acc = out_v[g, pl.ds(c * 16, 16)]
                out_v[g, pl.ds(c * 16, 16)] = (acc - npad_v * emb0_c) * inv_v

    pltpu.sync_copy(out_v, out_hbm.at[pl.ds(base, BPW)])


def _masked_mean_pool(tokens_t, emb):
    mesh = plsc.VectorSubcoreMesh(core_axis_name="c", subcore_axis_name="s")
    kern = pl.kernel(
        _pool_body,
        out_type=jax.ShapeDtypeStruct((B, EMB_DIM), jnp.float32),
        mesh=mesh,
        scratch_types=[
            pltpu.VMEM((L // 8, 8, BPW), jnp.int32),
            pltpu.VMEM((NBUF, BPW), jnp.int32),
            pltpu.VMEM((NBUF, BPW, EMB_DIM), jnp.float32),
            pltpu.VMEM((BPW, EMB_DIM), jnp.float32),
            pltpu.VMEM((16, EMB_DIM), jnp.float32),
            pltpu.VMEM((16,), jnp.int32),
        ] + [pltpu.SemaphoreType.DMA] * NBUF,
        compiler_params=pltpu.CompilerParams(use_tc_tiling_on_sc=False),
    )
    return kern(tokens_t, emb)


VOCAB = 1000000
DT_W = 32768
DT_H = DT_W // 2
DT_STEPS = -(-VOCAB // DT_W)
VOCAB_PAD = DT_STEPS * DT_W


def _detile_body(embt_ref, out_ref):
    x = embt_ref[...]
    ta = jnp.swapaxes(x[:, :DT_H], 0, 1)
    tb = jnp.swapaxes(x[:, DT_H:], 0, 1)
    out_ref[...] = jnp.concatenate([ta, tb], axis=1)


def _detile(emb_t):
    return pl.pallas_call(
        _detile_body,
        grid=(DT_STEPS,),
        in_specs=[pl.BlockSpec((EMB_DIM, DT_W), lambda k: (0, k))],
        out_specs=pl.BlockSpec((DT_H, 2 * EMB_DIM), lambda k: (k, 0)),
        out_shape=jax.ShapeDtypeStruct((VOCAB_PAD // 2, 2 * EMB_DIM),
                                       jnp.float32),
    )(emb_t)


def _mlp_body(avg_ref, w1_ref, b1_ref, w2_ref, b2_ref, out_ref):
    h = jnp.dot(avg_ref[...], w1_ref[...], preferred_element_type=jnp.float32)
    h = jnp.maximum(h + b1_ref[...], 0.0)
    out_ref[...] = (
        jnp.dot(h, w2_ref[...], preferred_element_type=jnp.float32)
        + b2_ref[...]
    )


def _mlp(avg, W1, b1, W2, b2):
    return pl.pallas_call(
        _mlp_body,
        out_shape=jax.ShapeDtypeStruct((B, OUT_DIM), jnp.float32),
    )(avg, W1, b1.reshape(1, -1), W2, b2.reshape(1, -1))


def kernel(tokens, emb, W1, b1, W2, b2):
    tokens_tiled = tokens.reshape(B // 128, 128, L // 8, 8).transpose(2, 0, 3, 1)
    emb_lin = _detile(emb.T).reshape(VOCAB_PAD, EMB_DIM)
    avg = _masked_mean_pool(tokens_tiled, emb_lin)
    logits = _mlp(avg, W1, b1, W2, b2)
    return logits.reshape(B, OUT_DIM // N_DISCRETE, N_DISCRETE)
